# baseline (speedup 1.0000x reference)
import functools

import jax
import jax.numpy as jnp
import numpy as np
from jax import lax
from jax.experimental import pallas as pl
from jax.experimental.pallas import tpu as pltpu
from jax.experimental.pallas import tpu_sc as plsc

_N = 10000
_E = 320000
_D = 128
_KP = 25

_NC = 2
_NS = 16
_HW = 64
_B = 64
_NB = 320
_EW = _B * _NB
_EPAD = _EW * _NS
_EALLOC = 328704
_EROWS = _EALLOC // 128
_NPAD = 10240
_RPT = _NPAD // _NS


def _tmat_body(x_ref, w_ref, o_ref):
    o_ref[0] = jnp.dot(
        x_ref[...], w_ref[0], preferred_element_type=jnp.float32)


def _tmat(x, wpair):
    xb = 2000
    return pl.pallas_call(
        _tmat_body,
        grid=(_N // xb, _KP),
        in_specs=[
            pl.BlockSpec((xb, _D), lambda nb, k: (nb, 0)),
            pl.BlockSpec((1, _D, 2 * _D), lambda nb, k: (k, 0, 0)),
        ],
        out_specs=pl.BlockSpec((1, xb, 2 * _D), lambda nb, k: (k, nb, 0)),
        out_shape=jax.ShapeDtypeStruct((_KP, _N, 2 * _D), jnp.float32),
    )(x, wpair)


def _prep_body(p0_ref, p1_ref, col_ref, row_ref, bas_ref, gidx_ref, dval_ref):
    v0 = p0_ref[...] * 4.0
    v1 = p1_ref[...] * 4.0
    f0 = jnp.floor(v0)
    f1 = jnp.floor(v1)
    fr0 = v0 - f0
    fr1 = v1 - f1
    i0 = f0.astype(jnp.int32)
    i1 = f1.astype(jnp.int32)
    col = col_ref[...]
    for s in range(4):
        b0 = s & 1
        b1 = (s >> 1) & 1
        bs = (fr0 if b0 else (1.0 - fr0)) * (fr1 if b1 else (1.0 - fr1))
        bas_ref[s] = bs
    gidx_ref[...] = (i1 * 5 + i0) * _N + col
    r = row_ref[...]
    lane = jnp.mod(lax.broadcasted_iota(jnp.int32, r.shape, 1), 16)
    cnt = jnp.ones(r.shape, jnp.float32)
    later = jnp.zeros(r.shape, jnp.bool_)
    for k in range(1, 16):
        prev = jnp.roll(r, k, axis=1)
        cnt = cnt + jnp.where((prev == r) & (lane >= k), 1.0, 0.0)
        nxt = jnp.roll(r, -k, axis=1)
        later = later | ((nxt == r) & (lane <= 15 - k))
    dval_ref[...] = jnp.where(later, 0.0, cnt)


def _prep(p0, p1, col, rowr):
    eb = 8
    return pl.pallas_call(
        _prep_body,
        grid=(_EROWS // eb,),
        in_specs=[
            pl.BlockSpec((eb, 128), lambda i: (i, 0)),
            pl.BlockSpec((eb, 128), lambda i: (i, 0)),
            pl.BlockSpec((eb, 128), lambda i: (i, 0)),
            pl.BlockSpec((eb, 128), lambda i: (i, 0)),
        ],
        out_specs=[
            pl.BlockSpec((4, eb, 128), lambda i: (0, i, 0)),
            pl.BlockSpec((eb, 128), lambda i: (i, 0)),
            pl.BlockSpec((eb, 128), lambda i: (i, 0)),
        ],
        out_shape=[
            jax.ShapeDtypeStruct((4, _EROWS, 128), jnp.float32),
            jax.ShapeDtypeStruct((_EROWS, 128), jnp.int32),
            jax.ShapeDtypeStruct((_EROWS, 128), jnp.float32),
        ],
    )(p0, p1, col, rowr)


def _sc_body(t_ref, gidx_ref, bas_ref, rowi_ref, dval_ref,
             out_ref, dout_ref, dpart_ref,
             rows, msg, gxb, gxb2, bsb, rib, dvb, sidx, degv, degred,
             acc,
             semf0, semf1, semf2, semf3,
             semg0, semg1, semg2, semg3, sems0, sems1):
    c = lax.axis_index("c")
    sid = lax.axis_index("s")
    ebase = sid * _EW
    cvec = jnp.full((16,), c, jnp.int32)

    semf = (semf0, semf1, semf2, semf3)
    semg = (semg0, semg1, semg2, semg3)
    sems = (sems0, sems1)

    zero16 = jnp.zeros((16,), jnp.float32)

    def _init_z(e, carry):
        for cc in range(_D // 16):
            msg[0, e, pl.ds(16 * cc, 16)] = zero16
            msg[1, e, pl.ds(16 * cc, 16)] = zero16
        return carry

    lax.fori_loop(0, _B, _init_z, 0)

    def _init_deg(i, carry):
        degv[pl.ds(i * 16, 16)] = zero16
        return carry

    lax.fori_loop(0, _NPAD // 16, _init_deg, 0)

    rpt2 = _NPAD // 2 // _NS
    for jz in range(rpt2 // _B):
        pltpu.sync_copy(msg.at[0], acc.at[pl.ds(sid * rpt2 + jz * _B, _B)])
    plsc.subcore_barrier()

    def _fetch(b, fs):
        base = ebase + b * _B
        pltpu.async_copy(gidx_ref.at[pl.ds(base, _B)], gxb.at[fs], semf[fs])
        for s in range(4):
            pltpu.async_copy(
                bas_ref.at[s, pl.ds(base, _B)], bsb.at[fs, s], semf[fs])
        pltpu.async_copy(rowi_ref.at[pl.ds(base, _B)], rib.at[fs], semf[fs])
        pltpu.async_copy(dval_ref.at[pl.ds(base, _B)], dvb.at[fs], semf[fs])

    def _wait_fetch(b, fs):
        base = ebase + b * _B
        pltpu.make_async_copy(
            gidx_ref.at[pl.ds(base, _B)], gxb.at[fs], semf[fs]).wait()
        for s in range(4):
            pltpu.make_async_copy(
                bas_ref.at[s, pl.ds(base, _B)], bsb.at[fs, s], semf[fs]).wait()
        pltpu.make_async_copy(
            rowi_ref.at[pl.ds(base, _B)], rib.at[fs], semf[fs]).wait()
        pltpu.make_async_copy(
            dval_ref.at[pl.ds(base, _B)], dvb.at[fs], semf[fs]).wait()
        for q in range(_B // 16):
            sl = pl.ds(q * 16, 16)
            v2 = gxb[fs, sl] * 2 + cvec
            gxb[fs, sl] = v2
            gxb2[fs, sl] = v2 + 10 * _N

    def _gathers(r, fs):
        pltpu.async_copy(t_ref.at[gxb.at[fs]], rows.at[r, 0], semg[r])
        pltpu.async_copy(t_ref.at[gxb2.at[fs]], rows.at[r, 1], semg[r])

    def _wait_gathers(r, fs):
        pltpu.make_async_copy(
            t_ref.at[gxb.at[fs]], rows.at[r, 0], semg[r]).wait()
        pltpu.make_async_copy(
            t_ref.at[gxb2.at[fs]], rows.at[r, 1], semg[r]).wait()

    def _scatter(p, fs):
        for q in range(_B // 16):
            sl = pl.ds(q * 16, 16)
            sidx[p, sl] = lax.shift_right_logical(rib[fs, sl], 1)
        pltpu.async_copy(msg.at[p], acc.at[sidx.at[p]], sems[p], add=True)

    def _wait_scatter(p):
        pltpu.make_async_copy(msg.at[p], acc.at[sidx.at[p]], sems[p]).wait()

    def _compute(p, fs):
        def _grp(g):
            gsl = pl.ds(g * 16, 16)

            @pl.when(c == 0)
            def _():
                rv = rib[fs, gsl]
                dv = dvb[fs, gsl]
                plsc.addupdate_scatter(degv, [rv], dv, mask=dv > 0.0)

            bv0 = bsb[fs, 0, gsl]
            bv1 = bsb[fs, 1, gsl]
            bv2 = bsb[fs, 2, gsl]
            bv3 = bsb[fs, 3, gsl]
            pvec = (rib[fs, gsl] & 1) * 64
            for l in range(16):
                e = g * 16 + l
                off = pvec[l]
                bb = [jnp.full((16,), bv0[l], jnp.float32),
                      jnp.full((16,), bv1[l], jnp.float32),
                      jnp.full((16,), bv2[l], jnp.float32),
                      jnp.full((16,), bv3[l], jnp.float32)]
                for jc in range(_HW // 16):
                    va = jnp.zeros((16,), jnp.float32)
                    for s2 in range(2):
                        for h in range(2):
                            sl = pl.ds(64 * h + 16 * jc, 16)
                            va = va + rows[p, s2, e, sl] * bb[2 * s2 + h]
                    msg[p, e, pl.ds(off + 16 * jc, 16)] = va
                    msg[p, e, pl.ds(64 - off + 16 * jc, 16)] = zero16
            return 0

        lax.fori_loop(0, _B // 16, lambda g, car: _grp(g), 0)

    _fetch(0, 0)
    _fetch(1, 1)
    _wait_fetch(0, 0)
    _gathers(0, 0)

    def _iter(i, carry):
        for j in range(4):
            b = i * 4 + j
            p = j % 2
            _wait_gathers(p, j)
            _wait_fetch(b + 1, (j + 1) % 4)
            _gathers(1 - p, (j + 1) % 4)

            @pl.when(b >= 2)
            def _():
                _wait_scatter(p)

            _fetch(b + 2, (j + 2) % 4)
            _compute(p, j)
            _scatter(p, j)
        return carry

    lax.fori_loop(0, _NB // 4, _iter, 0)

    _wait_gathers(0, 0)
    _wait_fetch(_NB + 1, 1)
    _wait_scatter(0)
    _wait_scatter(1)

    @pl.when(c == 0)
    def _():
        pltpu.sync_copy(degv, dpart_ref.at[sid])

    plsc.subcore_barrier()

    for qz in range(4):
        qr = rpt2 // 4
        pltpu.sync_copy(
            acc.at[pl.ds(sid * rpt2 + qz * qr, qr)],
            out_ref.at[c, pl.ds(sid * rpt2 + qz * qr, qr)])

    @pl.when(c == 0)
    def _():
        for t in range(_NS):
            pltpu.sync_copy(
                dpart_ref.at[t, pl.ds(sid * _RPT, _RPT)], degred.at[t])
        for i in range(_RPT // 16):
            v = degred[0, pl.ds(i * 16, 16)]
            for t in range(1, _NS):
                v = v + degred[t, pl.ds(i * 16, 16)]
            for l in range(16):
                degv[pl.ds((i * 16 + l) * 16, 16)] = jnp.full(
                    (16,), v[l], jnp.float32)
        pltpu.sync_copy(degv, dout_ref.at[pl.ds(sid * _RPT * 16, _RPT * 16)])


@functools.cache
def _make_sc_kernel():
    return pl.kernel(
        _sc_body,
        out_type=[
            jax.ShapeDtypeStruct((_NC, _NPAD // 2, _D), jnp.float32),
            jax.ShapeDtypeStruct((_NPAD * 16,), jnp.float32),
            jax.ShapeDtypeStruct((_NS, _NPAD), jnp.float32),
        ],
        mesh=plsc.VectorSubcoreMesh(core_axis_name="c", subcore_axis_name="s"),
        compiler_params=pltpu.CompilerParams(needs_layout_passes=False),
        scratch_types=[
            pltpu.VMEM((2, 2, _B, _D), jnp.float32),
            pltpu.VMEM((2, _B, _D), jnp.float32),
            pltpu.VMEM((4, _B), jnp.int32),
            pltpu.VMEM((4, _B), jnp.int32),
            pltpu.VMEM((4, 4, _B), jnp.float32),
            pltpu.VMEM((4, _B), jnp.int32),
            pltpu.VMEM((4, _B), jnp.float32),
            pltpu.VMEM((2, _B), jnp.int32),
            pltpu.VMEM((_NPAD,), jnp.float32),
            pltpu.VMEM((_NS, _RPT), jnp.float32),
            pltpu.VMEM_SHARED((_NPAD // 2, _D), jnp.float32),
            pltpu.SemaphoreType.DMA,
            pltpu.SemaphoreType.DMA,
            pltpu.SemaphoreType.DMA,
            pltpu.SemaphoreType.DMA,
            pltpu.SemaphoreType.DMA,
            pltpu.SemaphoreType.DMA,
            pltpu.SemaphoreType.DMA,
            pltpu.SemaphoreType.DMA,
            pltpu.SemaphoreType.DMA,
            pltpu.SemaphoreType.DMA,
        ],
    )


def _comb_body(acc_ref, deg_ref, x_ref, rw_ref, b_ref, o_ref):
    p = jnp.concatenate([acc_ref[0], acc_ref[1]], axis=-1)
    deg = jnp.maximum(deg_ref[:, 0:1], 1.0)
    out = p / deg
    out = out + jnp.dot(x_ref[...], rw_ref[...],
                        preferred_element_type=jnp.float32)
    o_ref[...] = out + b_ref[...]


def _combine(acc2, deg, x, root_weight, bias2d):
    cb = 2000
    return pl.pallas_call(
        _comb_body,
        grid=(_N // cb,),
        in_specs=[
            pl.BlockSpec((_NC, cb, _HW), lambda i: (0, i, 0)),
            pl.BlockSpec((cb, 16), lambda i: (i, 0)),
            pl.BlockSpec((cb, _D), lambda i: (i, 0)),
            pl.BlockSpec((_D, _D), lambda i: (0, 0)),
            pl.BlockSpec((1, _D), lambda i: (0, 0)),
        ],
        out_specs=pl.BlockSpec((cb, _D), lambda i: (i, 0)),
        out_shape=jax.ShapeDtypeStruct((_N, _D), jnp.float32),
    )(acc2, deg, x, root_weight, bias2d)


def kernel(x, edge_index, pseudo, weight, root_weight, bias):
    row = edge_index[0].astype(jnp.int32)
    col = edge_index[1].astype(jnp.int32)
    npadE = _EALLOC - _E
    row_p = jnp.concatenate([row, jnp.full((npadE,), _N, jnp.int32)])
    col_p = jnp.concatenate([col, jnp.zeros((npadE,), jnp.int32)])
    pseudo_p = jnp.concatenate(
        [pseudo, jnp.zeros((npadE, 2), pseudo.dtype)], axis=0)
    p0 = pseudo_p[:, 0].reshape(_EROWS, 128)
    p1 = pseudo_p[:, 1].reshape(_EROWS, 128)
    colr = col_p.reshape(_EROWS, 128)

    rowr = row_p.reshape(_EROWS, 128)

    wnext = jnp.roll(weight, -1, axis=0)
    wpair = jnp.concatenate(
        [weight[:, :, 0:_HW], wnext[:, :, 0:_HW],
         weight[:, :, _HW:_D], wnext[:, :, _HW:_D]],
        axis=2).astype(jnp.bfloat16)

    t3 = _tmat(x.astype(jnp.bfloat16), wpair).reshape(_KP * _N * 2, _D)

    bas4, gidx, dval = _prep(p0, p1, colr, rowr)
    bas4 = bas4.reshape(4, _EALLOC)
    gidx = gidx.reshape(_EALLOC)
    dval = dval.reshape(_EALLOC)

    acc2, deg, _dparts = _make_sc_kernel()(t3, gidx, bas4, row_p, dval)
    acc2 = acc2.reshape(_NC, _NPAD, _HW)
    deg = deg.reshape(_NPAD, 16)

    return _combine(acc2, deg, x, root_weight, bias.reshape(1, _D))


# speedup vs baseline: 1.2635x; 1.2635x over previous
import functools

import jax
import jax.numpy as jnp
import numpy as np
from jax import lax
from jax.experimental import pallas as pl
from jax.experimental.pallas import tpu as pltpu
from jax.experimental.pallas import tpu_sc as plsc

_N = 10000
_E = 320000
_D = 128
_KP = 25

_NC = 2
_NS = 16
_HW = 64
_B = 64
_NB = 320
_EW = _B * _NB
_EPAD = _EW * _NS
_EALLOC = 328704
_EROWS = _EALLOC // 128
_NPAD = 10240
_RPT = _NPAD // _NS


def _tmat_body(x_ref, w_ref, o_ref):
    o_ref[0] = jnp.dot(
        x_ref[...], w_ref[0], preferred_element_type=jnp.float32)


def _tmat_body2(x_ref, w_ref, o_ref):
    o_ref[0, 0] = jnp.dot(
        x_ref[...], w_ref[0, 0], preferred_element_type=jnp.float32)


def _tmat(x, wpair):
    xb = 2000
    return pl.pallas_call(
        _tmat_body2,
        grid=(_N // xb, _KP, _NC),
        in_specs=[
            pl.BlockSpec((xb, _D), lambda nb, k, c: (nb, 0)),
            pl.BlockSpec((1, 1, _D, _D), lambda nb, k, c: (c, k, 0, 0)),
        ],
        out_specs=pl.BlockSpec(
            (1, 1, xb, _D), lambda nb, k, c: (c, k, nb, 0)),
        out_shape=jax.ShapeDtypeStruct((_NC, _KP, _N, _D), jnp.float32),
    )(x, wpair)


def _prep_body(p0_ref, p1_ref, col_ref, row_ref, bas_ref, gidx_ref, dval_ref):
    v0 = p0_ref[...] * 4.0
    v1 = p1_ref[...] * 4.0
    f0 = jnp.floor(v0)
    f1 = jnp.floor(v1)
    fr0 = v0 - f0
    fr1 = v1 - f1
    i0 = f0.astype(jnp.int32)
    i1 = f1.astype(jnp.int32)
    col = col_ref[...]
    for s in range(4):
        b0 = s & 1
        b1 = (s >> 1) & 1
        bs = (fr0 if b0 else (1.0 - fr0)) * (fr1 if b1 else (1.0 - fr1))
        bas_ref[s] = bs
    gidx_ref[...] = (i1 * 5 + i0) * _N + col
    r = row_ref[...]
    lane = jnp.mod(lax.broadcasted_iota(jnp.int32, r.shape, 1), 16)
    cnt = jnp.ones(r.shape, jnp.float32)
    later = jnp.zeros(r.shape, jnp.bool_)
    for k in range(1, 16):
        prev = jnp.roll(r, k, axis=1)
        cnt = cnt + jnp.where((prev == r) & (lane >= k), 1.0, 0.0)
        nxt = jnp.roll(r, -k, axis=1)
        later = later | ((nxt == r) & (lane <= 15 - k))
    dval_ref[...] = jnp.where(later, 0.0, cnt)


def _prep(p0, p1, col, rowr):
    eb = 8
    return pl.pallas_call(
        _prep_body,
        grid=(_EROWS // eb,),
        in_specs=[
            pl.BlockSpec((eb, 128), lambda i: (i, 0)),
            pl.BlockSpec((eb, 128), lambda i: (i, 0)),
            pl.BlockSpec((eb, 128), lambda i: (i, 0)),
            pl.BlockSpec((eb, 128), lambda i: (i, 0)),
        ],
        out_specs=[
            pl.BlockSpec((4, eb, 128), lambda i: (0, i, 0)),
            pl.BlockSpec((eb, 128), lambda i: (i, 0)),
            pl.BlockSpec((eb, 128), lambda i: (i, 0)),
        ],
        out_shape=[
            jax.ShapeDtypeStruct((4, _EROWS, 128), jnp.float32),
            jax.ShapeDtypeStruct((_EROWS, 128), jnp.int32),
            jax.ShapeDtypeStruct((_EROWS, 128), jnp.float32),
        ],
    )(p0, p1, col, rowr)


def _sc_body(t_ref, gidx_ref, bas_ref, rowi_ref, dval_ref,
             out_ref, dout_ref, dpart_ref,
             rows, msg, gxb, gxb2, bsb, rib, dvb, sidx, degv, degred,
             acc,
             semf0, semf1, semf2, semf3,
             semg0, semg1, semg2, semg3, sems0, sems1):
    c = lax.axis_index("c")
    sid = lax.axis_index("s")
    ebase = sid * _EW
    cvec = jnp.full((16,), c * (_KP * _N), jnp.int32)

    semf = (semf0, semf1, semf2, semf3)
    semg = (semg0, semg1, semg2, semg3)
    sems = (sems0, sems1)

    zero16 = jnp.zeros((16,), jnp.float32)

    def _init_z(e, carry):
        for cc in range(_HW // 16):
            msg[0, e, pl.ds(16 * cc, 16)] = zero16
            msg[1, e, pl.ds(16 * cc, 16)] = zero16
        return carry

    lax.fori_loop(0, _B, _init_z, 0)

    def _init_deg(i, carry):
        degv[pl.ds(i * 16, 16)] = zero16
        return carry

    lax.fori_loop(0, _NPAD // 16, _init_deg, 0)

    for jz in range(_RPT // _B):
        pltpu.sync_copy(msg.at[0], acc.at[pl.ds(sid * _RPT + jz * _B, _B)])
    plsc.subcore_barrier()

    def _fetch(b, fs):
        base = ebase + b * _B
        pltpu.async_copy(gidx_ref.at[pl.ds(base, _B)], gxb.at[fs], semf[fs])
        for s in range(4):
            pltpu.async_copy(
                bas_ref.at[s, pl.ds(base, _B)], bsb.at[fs, s], semf[fs])
        pltpu.async_copy(rowi_ref.at[pl.ds(base, _B)], rib.at[fs], semf[fs])
        pltpu.async_copy(dval_ref.at[pl.ds(base, _B)], dvb.at[fs], semf[fs])

    def _wait_fetch(b, fs):
        base = ebase + b * _B
        pltpu.make_async_copy(
            gidx_ref.at[pl.ds(base, _B)], gxb.at[fs], semf[fs]).wait()
        for s in range(4):
            pltpu.make_async_copy(
                bas_ref.at[s, pl.ds(base, _B)], bsb.at[fs, s], semf[fs]).wait()
        pltpu.make_async_copy(
            rowi_ref.at[pl.ds(base, _B)], rib.at[fs], semf[fs]).wait()
        pltpu.make_async_copy(
            dval_ref.at[pl.ds(base, _B)], dvb.at[fs], semf[fs]).wait()
        for q in range(_B // 16):
            sl = pl.ds(q * 16, 16)
            v2 = gxb[fs, sl] + cvec
            gxb[fs, sl] = v2
            gxb2[fs, sl] = v2 + 5 * _N

    def _gathers(r, fs):
        pltpu.async_copy(t_ref.at[gxb.at[fs]], rows.at[r, 0], semg[r])
        pltpu.async_copy(t_ref.at[gxb2.at[fs]], rows.at[r, 1], semg[r])

    def _wait_gathers(r, fs):
        pltpu.make_async_copy(
            t_ref.at[gxb.at[fs]], rows.at[r, 0], semg[r]).wait()
        pltpu.make_async_copy(
            t_ref.at[gxb2.at[fs]], rows.at[r, 1], semg[r]).wait()

    def _scatter(p, fs):
        for q in range(_B // 16):
            sl = pl.ds(q * 16, 16)
            sidx[p, sl] = rib[fs, sl]
        pltpu.async_copy(msg.at[p], acc.at[sidx.at[p]], sems[p], add=True)

    def _wait_scatter(p):
        pltpu.make_async_copy(msg.at[p], acc.at[sidx.at[p]], sems[p]).wait()

    def _compute(p, fs):
        def _grp(g):
            gsl = pl.ds(g * 16, 16)

            @pl.when(c == 0)
            def _():
                rv = rib[fs, gsl]
                dv = dvb[fs, gsl]
                plsc.addupdate_scatter(degv, [rv], dv, mask=dv > 0.0)

            bv0 = bsb[fs, 0, gsl]
            bv1 = bsb[fs, 1, gsl]
            bv2 = bsb[fs, 2, gsl]
            bv3 = bsb[fs, 3, gsl]
            for l in range(16):
                e = g * 16 + l
                bb = [jnp.full((16,), bv0[l], jnp.float32),
                      jnp.full((16,), bv1[l], jnp.float32),
                      jnp.full((16,), bv2[l], jnp.float32),
                      jnp.full((16,), bv3[l], jnp.float32)]
                for jc in range(_HW // 16):
                    va = jnp.zeros((16,), jnp.float32)
                    for s2 in range(2):
                        for h in range(2):
                            sl = pl.ds(64 * h + 16 * jc, 16)
                            va = va + rows[p, s2, e, sl] * bb[2 * s2 + h]
                    msg[p, e, pl.ds(16 * jc, 16)] = va
            return 0

        lax.fori_loop(0, _B // 16, lambda g, car: _grp(g), 0)

    _fetch(0, 0)
    _fetch(1, 1)
    _wait_fetch(0, 0)
    _gathers(0, 0)

    def _iter(i, carry):
        for j in range(4):
            b = i * 4 + j
            p = j % 2
            _wait_gathers(p, j)
            _wait_fetch(b + 1, (j + 1) % 4)
            _gathers(1 - p, (j + 1) % 4)

            @pl.when(b >= 2)
            def _():
                _wait_scatter(p)

            _fetch(b + 2, (j + 2) % 4)
            _compute(p, j)
            _scatter(p, j)
        return carry

    lax.fori_loop(0, _NB // 4, _iter, 0)

    _wait_gathers(0, 0)
    _wait_fetch(_NB + 1, 1)
    _wait_scatter(0)
    _wait_scatter(1)

    @pl.when(c == 0)
    def _():
        pltpu.sync_copy(degv, dpart_ref.at[sid])

    plsc.subcore_barrier()

    for qz in range(4):
        qr = _RPT // 4
        pltpu.sync_copy(
            acc.at[pl.ds(sid * _RPT + qz * qr, qr)],
            out_ref.at[c, pl.ds(sid * _RPT + qz * qr, qr)])

    @pl.when(c == 0)
    def _():
        for t in range(_NS):
            pltpu.sync_copy(
                dpart_ref.at[t, pl.ds(sid * _RPT, _RPT)], degred.at[t])
        for i in range(_RPT // 16):
            v = degred[0, pl.ds(i * 16, 16)]
            for t in range(1, _NS):
                v = v + degred[t, pl.ds(i * 16, 16)]
            for l in range(16):
                degv[pl.ds((i * 16 + l) * 16, 16)] = jnp.full(
                    (16,), v[l], jnp.float32)
        pltpu.sync_copy(degv, dout_ref.at[pl.ds(sid * _RPT * 16, _RPT * 16)])


@functools.cache
def _make_sc_kernel():
    return pl.kernel(
        _sc_body,
        out_type=[
            jax.ShapeDtypeStruct((_NC, _NPAD, _HW), jnp.float32),
            jax.ShapeDtypeStruct((_NPAD * 16,), jnp.float32),
            jax.ShapeDtypeStruct((_NS, _NPAD), jnp.float32),
        ],
        mesh=plsc.VectorSubcoreMesh(core_axis_name="c", subcore_axis_name="s"),
        compiler_params=pltpu.CompilerParams(
            needs_layout_passes=False, use_tc_tiling_on_sc=False),
        scratch_types=[
            pltpu.VMEM((2, 2, _B, _D), jnp.float32),
            pltpu.VMEM((2, _B, _HW), jnp.float32),
            pltpu.VMEM((4, _B), jnp.int32),
            pltpu.VMEM((4, _B), jnp.int32),
            pltpu.VMEM((4, 4, _B), jnp.float32),
            pltpu.VMEM((4, _B), jnp.int32),
            pltpu.VMEM((4, _B), jnp.float32),
            pltpu.VMEM((2, _B), jnp.int32),
            pltpu.VMEM((_NPAD,), jnp.float32),
            pltpu.VMEM((_NS, _RPT), jnp.float32),
            pltpu.VMEM_SHARED((_NPAD, _HW), jnp.float32),
            pltpu.SemaphoreType.DMA,
            pltpu.SemaphoreType.DMA,
            pltpu.SemaphoreType.DMA,
            pltpu.SemaphoreType.DMA,
            pltpu.SemaphoreType.DMA,
            pltpu.SemaphoreType.DMA,
            pltpu.SemaphoreType.DMA,
            pltpu.SemaphoreType.DMA,
            pltpu.SemaphoreType.DMA,
            pltpu.SemaphoreType.DMA,
        ],
    )


def _comb_body(acc_ref, deg_ref, x_ref, rw_ref, b_ref, o_ref):
    p = jnp.concatenate([acc_ref[0], acc_ref[1]], axis=-1)
    deg = jnp.maximum(deg_ref[:, 0:1], 1.0)
    out = p / deg
    out = out + jnp.dot(x_ref[...], rw_ref[...],
                        preferred_element_type=jnp.float32)
    o_ref[...] = out + b_ref[...]


def _combine(acc2, deg, x, root_weight, bias2d):
    cb = 2000
    return pl.pallas_call(
        _comb_body,
        grid=(_N // cb,),
        in_specs=[
            pl.BlockSpec((_NC, cb, _HW), lambda i: (0, i, 0)),
            pl.BlockSpec((cb, 16), lambda i: (i, 0)),
            pl.BlockSpec((cb, _D), lambda i: (i, 0)),
            pl.BlockSpec((_D, _D), lambda i: (0, 0)),
            pl.BlockSpec((1, _D), lambda i: (0, 0)),
        ],
        out_specs=pl.BlockSpec((cb, _D), lambda i: (i, 0)),
        out_shape=jax.ShapeDtypeStruct((_N, _D), jnp.float32),
    )(acc2, deg, x, root_weight, bias2d)


def kernel(x, edge_index, pseudo, weight, root_weight, bias):
    row = edge_index[0].astype(jnp.int32)
    col = edge_index[1].astype(jnp.int32)
    npadE = _EALLOC - _E
    row_p = jnp.concatenate([row, jnp.full((npadE,), _N, jnp.int32)])
    col_p = jnp.concatenate([col, jnp.zeros((npadE,), jnp.int32)])
    pseudo_p = jnp.concatenate(
        [pseudo, jnp.zeros((npadE, 2), pseudo.dtype)], axis=0)
    p0 = pseudo_p[:, 0].reshape(_EROWS, 128)
    p1 = pseudo_p[:, 1].reshape(_EROWS, 128)
    colr = col_p.reshape(_EROWS, 128)

    rowr = row_p.reshape(_EROWS, 128)

    wnext = jnp.roll(weight, -1, axis=0)
    wpair = jnp.stack(
        [jnp.concatenate(
            [weight[:, :, c * _HW:(c + 1) * _HW],
             wnext[:, :, c * _HW:(c + 1) * _HW]], axis=2)
         for c in range(_NC)], axis=0).astype(jnp.bfloat16)

    t3 = _tmat(x.astype(jnp.bfloat16), wpair).reshape(_NC * _KP * _N, _D)

    bas4, gidx, dval = _prep(p0, p1, colr, rowr)
    bas4 = bas4.reshape(4, _EALLOC)
    gidx = gidx.reshape(_EALLOC)
    dval = dval.reshape(_EALLOC)

    acc2, deg, _dparts = _make_sc_kernel()(t3, gidx, bas4, row_p, dval)
    acc2 = acc2.reshape(_NC, _NPAD, _HW)
    deg = deg.reshape(_NPAD, 16)

    return _combine(acc2, deg, x, root_weight, bias.reshape(1, _D))


# speedup vs baseline: 2.0093x; 1.5902x over previous
import functools

import jax
import jax.numpy as jnp
from jax import lax
from jax.experimental import pallas as pl
from jax.experimental.pallas import tpu as pltpu
from jax.experimental.pallas import tpu_sc as plsc

_N = 10000
_E = 320000
_D = 128
_KP = 25

_NC = 2
_NS = 16
_HW = 64
_B = 64
_NB = 320
_EW = _B * _NB
_EPAD = _EW * _NS
_EALLOC = 328704
_EROWS = _EALLOC // 128
_NPAD = 10240
_RPT = _NPAD // _NS


def _tmat_body(x_ref, w_ref, o_ref):
    o_ref[0] = jnp.dot(x_ref[...], w_ref[0], preferred_element_type=jnp.float32)


def _tmat(x, weight):
    xb = 2000
    return pl.pallas_call(
        _tmat_body,
        grid=(_N // xb, _KP),
        in_specs=[
            pl.BlockSpec((xb, _D), lambda nb, k: (nb, 0)),
            pl.BlockSpec((1, _D, _D), lambda nb, k: (k, 0, 0)),
        ],
        out_specs=pl.BlockSpec((1, xb, _D), lambda nb, k: (k, nb, 0)),
        out_shape=jax.ShapeDtypeStruct((_KP, _N, _D), jnp.float32),
    )(x, weight)


def _prep_body(p0_ref, p1_ref, col_ref, row_ref, bas_ref, gidx_ref, dval_ref):
    v0 = p0_ref[...] * 4.0
    v1 = p1_ref[...] * 4.0
    f0 = jnp.floor(v0)
    f1 = jnp.floor(v1)
    fr0 = v0 - f0
    fr1 = v1 - f1
    i0 = f0.astype(jnp.int32)
    i1 = f1.astype(jnp.int32)
    col = col_ref[...]
    for s in range(4):
        b0 = s & 1
        b1 = (s >> 1) & 1
        bs = (fr0 if b0 else (1.0 - fr0)) * (fr1 if b1 else (1.0 - fr1))
        idx0 = jnp.mod(i0 + b0, 5)
        idx1 = jnp.mod(i1 + b1, 5)
        wi = idx0 + idx1 * 5
        bas_ref[s] = bs
        gidx_ref[s] = wi * _N + col
    r = row_ref[...]
    lane = jnp.mod(lax.broadcasted_iota(jnp.int32, r.shape, 1), 16)
    cnt = jnp.ones(r.shape, jnp.float32)
    later = jnp.zeros(r.shape, jnp.bool_)
    for k in range(1, 16):
        prev = jnp.roll(r, k, axis=1)
        cnt = cnt + jnp.where((prev == r) & (lane >= k), 1.0, 0.0)
        nxt = jnp.roll(r, -k, axis=1)
        later = later | ((nxt == r) & (lane <= 15 - k))
    dval_ref[...] = jnp.where(later, 0.0, cnt)


def _prep(p0, p1, col, rowr):
    eb = 8
    return pl.pallas_call(
        _prep_body,
        grid=(_EROWS // eb,),
        in_specs=[
            pl.BlockSpec((eb, 128), lambda i: (i, 0)),
            pl.BlockSpec((eb, 128), lambda i: (i, 0)),
            pl.BlockSpec((eb, 128), lambda i: (i, 0)),
            pl.BlockSpec((eb, 128), lambda i: (i, 0)),
        ],
        out_specs=[
            pl.BlockSpec((4, eb, 128), lambda i: (0, i, 0)),
            pl.BlockSpec((4, eb, 128), lambda i: (0, i, 0)),
            pl.BlockSpec((eb, 128), lambda i: (i, 0)),
        ],
        out_shape=[
            jax.ShapeDtypeStruct((4, _EROWS, 128), jnp.float32),
            jax.ShapeDtypeStruct((4, _EROWS, 128), jnp.int32),
            jax.ShapeDtypeStruct((_EROWS, 128), jnp.float32),
        ],
    )(p0, p1, col, rowr)


def _sc_body(t_ref, gidx_ref, bas_ref, rowi_ref, dval_ref, out_ref, dout_ref,
             rows, msg, gxb, bsb, rib, dvb, degv, degred,
             acc, degsh,
             semf0, semf1, semf2, semf3, semg0, semg1, sems0, sems1):
    c = lax.axis_index("c")
    sid = lax.axis_index("s")
    ebase = sid * _EW
    cvec = jnp.full((16,), c, jnp.int32)

    semf = (semf0, semf1, semf2, semf3)
    semg = (semg0, semg1)
    sems = (sems0, sems1)

    zero16 = jnp.zeros((16,), jnp.float32)

    def _init_z(e, carry):
        for cc in range(_HW // 16):
            msg[0, e, pl.ds(16 * cc, 16)] = zero16
            msg[1, e, pl.ds(16 * cc, 16)] = zero16
        return carry

    lax.fori_loop(0, _B, _init_z, 0)

    def _init_deg(i, carry):
        degv[i, pl.ds(0, 16)] = zero16
        return carry

    lax.fori_loop(0, _NPAD // 16, _init_deg, 0)

    for jz in range(_RPT // _B):
        pltpu.sync_copy(msg.at[0], acc.at[pl.ds(sid * _RPT + jz * _B, _B)])
    plsc.subcore_barrier()

    def _fetch(b, fs):
        base = ebase + b * _B
        for s in range(4):
            pltpu.async_copy(
                gidx_ref.at[s, pl.ds(base, _B)], gxb.at[fs, s], semf[fs])
            pltpu.async_copy(
                bas_ref.at[s, pl.ds(base, _B)], bsb.at[fs, s], semf[fs])
        pltpu.async_copy(rowi_ref.at[pl.ds(base, _B)], rib.at[fs], semf[fs])
        pltpu.async_copy(dval_ref.at[pl.ds(base, _B)], dvb.at[fs], semf[fs])

    def _wait_fetch(b, fs):
        base = ebase + b * _B
        for s in range(4):
            pltpu.make_async_copy(
                gidx_ref.at[s, pl.ds(base, _B)], gxb.at[fs, s], semf[fs]).wait()
            pltpu.make_async_copy(
                bas_ref.at[s, pl.ds(base, _B)], bsb.at[fs, s], semf[fs]).wait()
        pltpu.make_async_copy(
            rowi_ref.at[pl.ds(base, _B)], rib.at[fs], semf[fs]).wait()
        pltpu.make_async_copy(
            dval_ref.at[pl.ds(base, _B)], dvb.at[fs], semf[fs]).wait()
        for s in range(4):
            for q in range(_B // 16):
                sl = pl.ds(q * 16, 16)
                gxb[fs, s, sl] = gxb[fs, s, sl] * 2 + cvec

    def _gathers(p, fs):
        for s in range(4):
            pltpu.async_copy(t_ref.at[gxb.at[fs, s]], rows.at[p, s], semg[p])

    def _wait_gathers(p, fs):
        for s in range(4):
            pltpu.make_async_copy(
                t_ref.at[gxb.at[fs, s]], rows.at[p, s], semg[p]).wait()

    def _scatter(p, fs):
        pltpu.async_copy(msg.at[p], acc.at[rib.at[fs]], sems[p], add=True)

    def _wait_scatter(p, fs):
        pltpu.make_async_copy(msg.at[p], acc.at[rib.at[fs]], sems[p]).wait()

    def _compute(p, fs):
        def _grp(g):
            gsl = pl.ds(g * 16, 16)

            @pl.when(c == 0)
            def _():
                rv = rib[fs, gsl]
                dv = dvb[fs, gsl]
                plsc.addupdate_scatter(
                    degv, [lax.shift_right_logical(rv, 4), rv & 15],
                    dv, mask=dv > 0.0)

            bv0 = bsb[fs, 0, gsl]
            bv1 = bsb[fs, 1, gsl]
            bv2 = bsb[fs, 2, gsl]
            bv3 = bsb[fs, 3, gsl]
            for l in range(16):
                e = g * 16 + l
                b0 = jnp.full((16,), bv0[l], jnp.float32)
                b1 = jnp.full((16,), bv1[l], jnp.float32)
                b2 = jnp.full((16,), bv2[l], jnp.float32)
                b3 = jnp.full((16,), bv3[l], jnp.float32)
                for jc in range(_HW // 16):
                    sl = pl.ds(16 * jc, 16)
                    v = (rows[p, 0, e, sl] * b0 + rows[p, 1, e, sl] * b1
                         + rows[p, 2, e, sl] * b2 + rows[p, 3, e, sl] * b3)
                    msg[p, e, sl] = v
            return 0

        lax.fori_loop(0, _B // 16, lambda g, car: _grp(g), 0)

    _fetch(0, 0)
    _fetch(1, 1)
    _wait_fetch(0, 0)
    _gathers(0, 0)

    def _iter(i, carry):
        for j in range(4):
            b = i * 4 + j
            p = j % 2
            _wait_gathers(p, j)
            _wait_fetch(b + 1, (j + 1) % 4)
            _gathers(1 - p, (j + 1) % 4)

            @pl.when(b >= 2)
            def _():
                _wait_scatter(p, (j + 2) % 4)

            _fetch(b + 2, (j + 2) % 4)
            _compute(p, j)
            _scatter(p, j)
        return carry

    lax.fori_loop(0, _NB // 4, _iter, 0)

    _wait_fetch(_NB + 1, 1)
    _wait_gathers(0, 0)
    _wait_scatter(0, 2)
    _wait_scatter(1, 3)

    @pl.when(c == 0)
    def _():
        pltpu.sync_copy(degv, degsh.at[sid])

    plsc.subcore_barrier()

    pltpu.sync_copy(acc.at[pl.ds(sid * _RPT, _RPT)],
                    out_ref.at[c, pl.ds(sid * _RPT, _RPT)])

    @pl.when(c == 0)
    def _():
        nr = _RPT // 16
        for t in range(_NS):
            pltpu.sync_copy(degsh.at[t, pl.ds(sid * nr, nr)], degred.at[t])
        for i in range(nr):
            v = degred[0, i, pl.ds(0, 16)]
            for t in range(1, _NS):
                v = v + degred[t, i, pl.ds(0, 16)]
            for l in range(16):
                degv[i * 16 + l, pl.ds(0, 16)] = jnp.full(
                    (16,), v[l], jnp.float32)
        pltpu.sync_copy(degv, dout_ref.at[pl.ds(sid * _RPT, _RPT)])


@functools.cache
def _make_sc_kernel():
    return pl.kernel(
        _sc_body,
        out_type=[
            jax.ShapeDtypeStruct((_NC, _NPAD, _HW), jnp.float32),
            jax.ShapeDtypeStruct((_NPAD, 16), jnp.float32),
        ],
        mesh=plsc.VectorSubcoreMesh(core_axis_name="c", subcore_axis_name="s"),
        compiler_params=pltpu.CompilerParams(
            needs_layout_passes=False, use_tc_tiling_on_sc=False),
        scratch_types=[
            pltpu.VMEM((2, 4, _B, _HW), jnp.float32),
            pltpu.VMEM((2, _B, _HW), jnp.float32),
            pltpu.VMEM((4, 4, _B), jnp.int32),
            pltpu.VMEM((4, 4, _B), jnp.float32),
            pltpu.VMEM((4, _B), jnp.int32),
            pltpu.VMEM((4, _B), jnp.float32),
            pltpu.VMEM((_NPAD // 16, 16), jnp.float32),
            pltpu.VMEM((_NS, _RPT // 16, 16), jnp.float32),
            pltpu.VMEM_SHARED((_NPAD, _HW), jnp.float32),
            pltpu.VMEM_SHARED((_NS, _NPAD // 16, 16), jnp.float32),
            pltpu.SemaphoreType.DMA,
            pltpu.SemaphoreType.DMA,
            pltpu.SemaphoreType.DMA,
            pltpu.SemaphoreType.DMA,
            pltpu.SemaphoreType.DMA,
            pltpu.SemaphoreType.DMA,
            pltpu.SemaphoreType.DMA,
            pltpu.SemaphoreType.DMA,
        ],
    )


def _comb_body(acc_ref, deg_ref, x_ref, rw_ref, b_ref, o_ref):
    p = jnp.concatenate([acc_ref[0], acc_ref[1]], axis=-1)
    deg = jnp.maximum(deg_ref[:, 0:1], 1.0)
    out = p / deg
    out = out + jnp.dot(x_ref[...], rw_ref[...],
                        preferred_element_type=jnp.float32)
    o_ref[...] = out + b_ref[...]


def _combine(acc2, deg, x, root_weight, bias2d):
    cb = 2000
    return pl.pallas_call(
        _comb_body,
        grid=(_N // cb,),
        in_specs=[
            pl.BlockSpec((_NC, cb, _HW), lambda i: (0, i, 0)),
            pl.BlockSpec((cb, 16), lambda i: (i, 0)),
            pl.BlockSpec((cb, _D), lambda i: (i, 0)),
            pl.BlockSpec((_D, _D), lambda i: (0, 0)),
            pl.BlockSpec((1, _D), lambda i: (0, 0)),
        ],
        out_specs=pl.BlockSpec((cb, _D), lambda i: (i, 0)),
        out_shape=jax.ShapeDtypeStruct((_N, _D), jnp.float32),
    )(acc2, deg, x, root_weight, bias2d)


def kernel(x, edge_index, pseudo, weight, root_weight, bias):
    row = edge_index[0].astype(jnp.int32)
    col = edge_index[1].astype(jnp.int32)
    npadE = _EALLOC - _E
    row_p = jnp.concatenate([row, jnp.full((npadE,), _N, jnp.int32)])
    col_p = jnp.concatenate([col, jnp.zeros((npadE,), jnp.int32)])
    pseudo_p = jnp.concatenate(
        [pseudo, jnp.zeros((npadE, 2), pseudo.dtype)], axis=0)
    p0 = pseudo_p[:, 0].reshape(_EROWS, 128)
    p1 = pseudo_p[:, 1].reshape(_EROWS, 128)
    colr = col_p.reshape(_EROWS, 128)

    rowr = row_p.reshape(_EROWS, 128)

    t = _tmat(x, weight).reshape(_KP * _N * 2, _HW)
    bas4, gidx4, dval = _prep(p0, p1, colr, rowr)
    bas4 = bas4.reshape(4, _EALLOC)
    gidx4 = gidx4.reshape(4, _EALLOC)
    dval = dval.reshape(_EALLOC)

    acc2, deg = _make_sc_kernel()(t, gidx4, bas4, row_p, dval)

    return _combine(acc2, deg, x, root_weight, bias.reshape(1, _D))
